# Initial kernel scaffold; baseline (speedup 1.0000x reference)
import jax
import jax.numpy as jnp
from jax.experimental import pallas as pl


def kernel(X, H, C, M, T):
    raise NotImplementedError("write your pallas kernel here")


# speedup vs baseline: 34.1704x; 34.1704x over previous
import functools

import jax
import jax.numpy as jnp
from jax.experimental import pallas as pl


def _loss_kernel(x_ref, h_ref, c_ref, m_ref, out_ref, *, T, d):
    H = h_ref[...]
    M = m_ref[...]
    dims = (((1,), (1,)), ((), ()))
    G = jax.lax.dot_general(H, H, dims, preferred_element_type=jnp.float32)
    GM = jax.lax.dot_general(M, M, dims, preferred_element_type=jnp.float32)
    nrm = jnp.sum(H * H, axis=1, keepdims=True)
    s = jnp.sum(H, axis=1, keepdims=True)
    mn = jnp.sum(M, axis=1, keepdims=True)

    sqd = nrm + nrm.T - 2.0 * G
    ds = s - s.T
    var = (sqd - ds * ds * (1.0 / d)) * (1.0 / (d - 1.0))
    good = var > 0.0
    score = jnp.where(good, jnp.sqrt(jnp.where(good, var, 1.0)), 0.0)

    msq = mn + mn.T - 2.0 * GM
    iota_r = jax.lax.broadcasted_iota(jnp.int32, (T, T), 0)
    iota_c = jax.lax.broadcasted_iota(jnp.int32, (T, T), 1)
    invalid = (iota_r == iota_c) | (msq <= 0.5)
    work = jnp.where(invalid, jnp.float32(9999.0), score)

    v0 = None
    num = jnp.zeros((T, 1), jnp.float32)
    den = jnp.zeros((T, 1), jnp.float32)
    for _ in range(8):
        v = jnp.min(work, axis=1, keepdims=True)
        is_min = work == v
        cand = jnp.where(is_min, iota_c, T)
        am = jnp.min(cand, axis=1, keepdims=True)
        chosen = iota_c == am
        sq_sel = jnp.sum(jnp.where(chosen, sqd, 0.0), axis=1, keepdims=True)
        work = jnp.where(chosen, jnp.float32(jnp.inf), work)
        if v0 is None:
            v0 = v
        e = jnp.exp(v0 - v)
        goodn = sq_sel > 0.0
        norm = jnp.where(goodn, jnp.sqrt(jnp.where(goodn, sq_sel, 1.0)), 0.0)
        num = num + e * norm
        den = den + e
    row_loss = jnp.sum(num / den)

    dd = x_ref[...] - H + c_ref[...]
    mse = jnp.sum(M * dd * dd)
    out_ref[...] = jnp.reshape(mse + row_loss, (1, 1))


def kernel(X, H, C, M, T):
    del T
    T, d = H.shape
    out = pl.pallas_call(
        functools.partial(_loss_kernel, T=T, d=d),
        out_shape=jax.ShapeDtypeStruct((1, 1), jnp.float32),
    )(X, H, C, M)
    return out[0, 0]
